# baseline (speedup 1.0000x reference)
import jax
import jax.numpy as jnp
from jax import lax
from jax.experimental import pallas as pl
from jax.experimental.pallas import tpu as pltpu
from jax.experimental.pallas import tpu_sc as plsc

N_NODES = 10000
N_EDGES = 320000
D_IN = 128
D_OUT = 128

NC = 2
NS = 16
NW = NC * NS
CH = 128
NCH = 80
EPW = NCH * CH
E_PAD = EPW * NW - N_EDGES
NP = 10016
RPT = 624
REM = NP - NS * RPT
ZR = 48


def _sc_aggregate_body(feat_hbm, srcp_hbm, dstp_hbm, parts_hbm, degp_hbm,
                       s0, s1, s2, s3, d0, d1, d2, d3, jidx,
                       rows0, rows1, deg_v, agg,
                       i0, i1, i2, i3, gsem0, gsem1, ssem0, ssem1):
    cid = lax.axis_index("c")
    sid = lax.axis_index("s")
    wid = cid * NS + sid

    zvec = jnp.zeros((16,), jnp.float32)
    jvec = jnp.full((16,), N_NODES, jnp.int32)
    for j in range(CH // 16):
        jidx[pl.ds(j * 16, 16)] = jvec

    def _zero_row(i, _):
        for j in range(D_IN // 16):
            rows0[i, pl.ds(j * 16, 16)] = zvec
        return 0

    lax.fori_loop(0, ZR, _zero_row, 0)

    def _zero_deg(i, _):
        deg_v[pl.ds(i * 16, 16)] = zvec
        return 0

    lax.fori_loop(0, NP // 16, _zero_deg, 0)

    for k in range(RPT // ZR):
        pltpu.sync_copy(rows0.at[pl.ds(0, ZR)],
                        agg.at[pl.ds(sid * RPT + k * ZR, ZR)])

    @pl.when(sid == NS - 1)
    def _zero_rem():
        pltpu.sync_copy(rows0.at[pl.ds(0, REM)], agg.at[pl.ds(NS * RPT, REM)])

    plsc.subcore_barrier()

    ones16 = jnp.ones((16,), jnp.float32)
    idxs = ((s0, d0, i0), (s1, d1, i1), (s2, d2, i2), (s3, d3, i3))
    rbuf = ((rows0, gsem0, ssem0), (rows1, gsem1, ssem1))

    pltpu.sync_copy(srcp_hbm.at[wid, 0], s0)
    pltpu.sync_copy(dstp_hbm.at[wid, 0], d0)
    pltpu.async_copy(srcp_hbm.at[wid, 1], s1, i1)
    pltpu.async_copy(dstp_hbm.at[wid, 1], d1, i1)
    pltpu.async_copy(rows0, agg.at[jidx], ssem1, add=True)
    pltpu.async_copy(feat_hbm.at[s0], rows0, gsem0)

    def _quad(t, _):
        for b in range(4):
            c = 4 * t + b
            sidx_c, didx_c, _ = idxs[b]
            sidx_n, didx_n, isem_n = idxs[(b + 1) % 4]
            sidx_l, didx_l, isem_l = idxs[(b + 2) % 4]
            rows_p, gsem_p, ssem_p = rbuf[b % 2]
            rows_q, gsem_q, ssem_q = rbuf[1 - b % 2]
            for g in range(CH // 16):
                d16 = didx_c[pl.ds(g * 16, 16)]
                s16 = sidx_c[pl.ds(g * 16, 16)]
                plsc.addupdate_scatter(deg_v, [d16], ones16,
                                       mask=s16 < N_NODES)
            pltpu.async_copy(srcp_hbm.at[wid, c + 2], sidx_l, isem_l)
            pltpu.async_copy(dstp_hbm.at[wid, c + 2], didx_l, isem_l)
            pltpu.make_async_copy(rows_q, agg.at[jidx], ssem_q).wait()
            pltpu.make_async_copy(srcp_hbm.at[wid, c], sidx_n, isem_n).wait()
            pltpu.make_async_copy(dstp_hbm.at[wid, c], didx_n, isem_n).wait()
            pltpu.async_copy(feat_hbm.at[sidx_n], rows_q, gsem_q)
            pltpu.make_async_copy(feat_hbm.at[sidx_c], rows_p, gsem_p).wait()
            pltpu.async_copy(rows_p, agg.at[didx_c], ssem_p, add=True)
        return 0

    lax.fori_loop(0, NCH // 4, _quad, 0)

    pltpu.make_async_copy(feat_hbm.at[s0], rows0, gsem0).wait()
    pltpu.make_async_copy(rows1, agg.at[jidx], ssem1).wait()
    pltpu.make_async_copy(srcp_hbm.at[wid, 0], s1, i1).wait()
    pltpu.make_async_copy(dstp_hbm.at[wid, 0], d1, i1).wait()

    plsc.subcore_barrier()

    pltpu.sync_copy(agg.at[pl.ds(sid * RPT, RPT)],
                    parts_hbm.at[cid, pl.ds(sid * RPT, RPT)])

    @pl.when(sid == NS - 1)
    def _write_rem():
        pltpu.sync_copy(agg.at[pl.ds(NS * RPT, REM)],
                        parts_hbm.at[cid, pl.ds(NS * RPT, REM)])

    pltpu.sync_copy(deg_v, degp_hbm.at[wid])


def _sc_aggregate(feat_p, srcp, dstp):
    mesh = plsc.VectorSubcoreMesh(core_axis_name="c", subcore_axis_name="s")
    return pl.kernel(
        _sc_aggregate_body,
        out_type=(jax.ShapeDtypeStruct((NC, NP, D_IN), jnp.float32),
                  jax.ShapeDtypeStruct((NW, NP), jnp.float32)),
        mesh=mesh,
        compiler_params=pltpu.CompilerParams(needs_layout_passes=False),
        scratch_types=[
            pltpu.VMEM((CH,), jnp.int32),
            pltpu.VMEM((CH,), jnp.int32),
            pltpu.VMEM((CH,), jnp.int32),
            pltpu.VMEM((CH,), jnp.int32),
            pltpu.VMEM((CH,), jnp.int32),
            pltpu.VMEM((CH,), jnp.int32),
            pltpu.VMEM((CH,), jnp.int32),
            pltpu.VMEM((CH,), jnp.int32),
            pltpu.VMEM((CH,), jnp.int32),
            pltpu.VMEM((CH, D_IN), jnp.float32),
            pltpu.VMEM((CH, D_IN), jnp.float32),
            pltpu.VMEM((NP,), jnp.float32),
            pltpu.VMEM_SHARED((NP, D_IN), jnp.float32),
            pltpu.SemaphoreType.DMA,
            pltpu.SemaphoreType.DMA,
            pltpu.SemaphoreType.DMA,
            pltpu.SemaphoreType.DMA,
            pltpu.SemaphoreType.DMA,
            pltpu.SemaphoreType.DMA,
            pltpu.SemaphoreType.DMA,
            pltpu.SemaphoreType.DMA,
        ],
    )(feat_p, srcp, dstp)


BR = 1000


def _tc_epilogue_body(feat_ref, parts_ref, degp_ref, ws_ref, wn_ref, b_ref,
                      out_ref):
    agg = parts_ref[0] + parts_ref[1]
    deg = jnp.sum(degp_ref[...], axis=1)[:, None]
    h_neigh = agg / jnp.maximum(deg, 1.0)
    acc = jnp.dot(feat_ref[...], ws_ref[...], preferred_element_type=jnp.float32)
    acc = acc + jnp.dot(h_neigh, wn_ref[...], preferred_element_type=jnp.float32)
    out_ref[...] = jnp.maximum(acc + b_ref[...], 0.0)


def _tc_epilogue(feat, parts, deg_parts_t, W_self, W_neigh, b2d):
    return pl.pallas_call(
        _tc_epilogue_body,
        grid=(N_NODES // BR,),
        in_specs=[
            pl.BlockSpec((BR, D_IN), lambda i: (i, 0)),
            pl.BlockSpec((NC, BR, D_IN), lambda i: (0, i, 0)),
            pl.BlockSpec((BR, NW), lambda i: (i, 0)),
            pl.BlockSpec((D_IN, D_OUT), lambda i: (0, 0)),
            pl.BlockSpec((D_IN, D_OUT), lambda i: (0, 0)),
            pl.BlockSpec((1, D_OUT), lambda i: (0, 0)),
        ],
        out_specs=pl.BlockSpec((BR, D_OUT), lambda i: (i, 0)),
        out_shape=jax.ShapeDtypeStruct((N_NODES, D_OUT), jnp.float32),
    )(feat, parts, deg_parts_t, W_self, W_neigh, b2d)


@jax.jit
def kernel(feat, edge_index, W_self, W_neigh, b):
    src = edge_index[0].astype(jnp.int32)
    dst = edge_index[1].astype(jnp.int32)
    feat_p = jnp.pad(feat, ((0, 8), (0, 0)))
    srcp = jnp.concatenate(
        [src, jnp.full((E_PAD,), N_NODES, jnp.int32)]).reshape(NW, NCH, CH)
    dstp = jnp.concatenate(
        [dst, jnp.arange(E_PAD, dtype=jnp.int32) % NP]).reshape(NW, NCH, CH)
    srcp = jnp.pad(srcp, ((0, 0), (0, 2), (0, 0)),
                   constant_values=N_NODES)
    dstp = jnp.pad(dstp, ((0, 0), (0, 2), (0, 0)))
    parts, deg_parts = _sc_aggregate(feat_p, srcp, dstp)
    return _tc_epilogue(feat, parts, deg_parts.T, W_self, W_neigh,
                        b.reshape(1, D_OUT))


# speedup vs baseline: 1.0443x; 1.0443x over previous
import jax
import jax.numpy as jnp
from jax import lax
from jax.experimental import pallas as pl
from jax.experimental.pallas import tpu as pltpu
from jax.experimental.pallas import tpu_sc as plsc

N_NODES = 10000
N_EDGES = 320000
D_IN = 128
D_OUT = 128

NC = 2
NS = 16
NW = NC * NS
CH = 128
NCH = 80
EPW = NCH * CH
E_PAD = EPW * NW - N_EDGES
NP = 10016
RPT = 624
REM = NP - NS * RPT
ZR = 48


def _sc_aggregate_body(feat_hbm, srcp_hbm, dstp_hbm, parts_hbm, degp_hbm,
                       s0, s1, s2, s3, d0, d1, d2, d3, jidx,
                       rows0, rows1, deg_v, agg,
                       i0, i1, i2, i3, gsem0, gsem1, ssem0, ssem1):
    cid = lax.axis_index("c")
    sid = lax.axis_index("s")
    wid = cid * NS + sid

    zvec = jnp.zeros((16,), jnp.float32)
    jvec = jnp.full((16,), N_NODES, jnp.int32)
    for j in range(CH // 16):
        jidx[pl.ds(j * 16, 16)] = jvec

    def _zero_row(i, _):
        for j in range(D_IN // 16):
            rows0[i, pl.ds(j * 16, 16)] = zvec
        return 0

    lax.fori_loop(0, ZR, _zero_row, 0)

    def _zero_deg(i, _):
        deg_v[pl.ds(i * 16, 16)] = zvec
        return 0

    lax.fori_loop(0, NP // 16, _zero_deg, 0)

    for k in range(RPT // ZR):
        pltpu.sync_copy(rows0.at[pl.ds(0, ZR)],
                        agg.at[pl.ds(sid * RPT + k * ZR, ZR)])

    @pl.when(sid == NS - 1)
    def _zero_rem():
        pltpu.sync_copy(rows0.at[pl.ds(0, REM)], agg.at[pl.ds(NS * RPT, REM)])

    plsc.subcore_barrier()

    ones16 = jnp.ones((16,), jnp.float32)

    def _chunk(c, _):
        pltpu.sync_copy(srcp_hbm.at[wid, c], s0)
        pltpu.sync_copy(dstp_hbm.at[wid, c], d0)
        gat = pltpu.async_copy(feat_hbm.at[s0], rows0, gsem0)
        for g in range(CH // 16):
            d16 = d0[pl.ds(g * 16, 16)]
            s16 = s0[pl.ds(g * 16, 16)]
            plsc.addupdate_scatter(deg_v, [d16], ones16, mask=s16 < N_NODES)
        gat.wait()
        pltpu.sync_copy(rows0, agg.at[d0], add=True)
        return 0

    lax.fori_loop(0, NCH, _chunk, 0)

    plsc.subcore_barrier()

    pltpu.sync_copy(agg.at[pl.ds(sid * RPT, RPT)],
                    parts_hbm.at[cid, pl.ds(sid * RPT, RPT)])

    @pl.when(sid == NS - 1)
    def _write_rem():
        pltpu.sync_copy(agg.at[pl.ds(NS * RPT, REM)],
                        parts_hbm.at[cid, pl.ds(NS * RPT, REM)])

    pltpu.sync_copy(deg_v, degp_hbm.at[wid])


def _sc_aggregate(feat_p, srcp, dstp):
    mesh = plsc.VectorSubcoreMesh(core_axis_name="c", subcore_axis_name="s")
    return pl.kernel(
        _sc_aggregate_body,
        out_type=(jax.ShapeDtypeStruct((NC, NP, D_IN), jnp.float32),
                  jax.ShapeDtypeStruct((NW, NP), jnp.float32)),
        mesh=mesh,
        compiler_params=pltpu.CompilerParams(needs_layout_passes=False),
        scratch_types=[
            pltpu.VMEM((CH,), jnp.int32),
            pltpu.VMEM((CH,), jnp.int32),
            pltpu.VMEM((CH,), jnp.int32),
            pltpu.VMEM((CH,), jnp.int32),
            pltpu.VMEM((CH,), jnp.int32),
            pltpu.VMEM((CH,), jnp.int32),
            pltpu.VMEM((CH,), jnp.int32),
            pltpu.VMEM((CH,), jnp.int32),
            pltpu.VMEM((CH,), jnp.int32),
            pltpu.VMEM((CH, D_IN), jnp.float32),
            pltpu.VMEM((CH, D_IN), jnp.float32),
            pltpu.VMEM((NP,), jnp.float32),
            pltpu.VMEM_SHARED((NP, D_IN), jnp.float32),
            pltpu.SemaphoreType.DMA,
            pltpu.SemaphoreType.DMA,
            pltpu.SemaphoreType.DMA,
            pltpu.SemaphoreType.DMA,
            pltpu.SemaphoreType.DMA,
            pltpu.SemaphoreType.DMA,
            pltpu.SemaphoreType.DMA,
            pltpu.SemaphoreType.DMA,
        ],
    )(feat_p, srcp, dstp)


BR = 1000


def _tc_epilogue_body(feat_ref, parts_ref, degp_ref, ws_ref, wn_ref, b_ref,
                      out_ref):
    agg = parts_ref[0] + parts_ref[1]
    deg = jnp.sum(degp_ref[...], axis=1)[:, None]
    h_neigh = agg / jnp.maximum(deg, 1.0)
    acc = jnp.dot(feat_ref[...], ws_ref[...], preferred_element_type=jnp.float32)
    acc = acc + jnp.dot(h_neigh, wn_ref[...], preferred_element_type=jnp.float32)
    out_ref[...] = jnp.maximum(acc + b_ref[...], 0.0)


def _tc_epilogue(feat, parts, deg_parts_t, W_self, W_neigh, b2d):
    return pl.pallas_call(
        _tc_epilogue_body,
        grid=(N_NODES // BR,),
        in_specs=[
            pl.BlockSpec((BR, D_IN), lambda i: (i, 0)),
            pl.BlockSpec((NC, BR, D_IN), lambda i: (0, i, 0)),
            pl.BlockSpec((BR, NW), lambda i: (i, 0)),
            pl.BlockSpec((D_IN, D_OUT), lambda i: (0, 0)),
            pl.BlockSpec((D_IN, D_OUT), lambda i: (0, 0)),
            pl.BlockSpec((1, D_OUT), lambda i: (0, 0)),
        ],
        out_specs=pl.BlockSpec((BR, D_OUT), lambda i: (i, 0)),
        out_shape=jax.ShapeDtypeStruct((N_NODES, D_OUT), jnp.float32),
    )(feat, parts, deg_parts_t, W_self, W_neigh, b2d)


@jax.jit
def kernel(feat, edge_index, W_self, W_neigh, b):
    src = edge_index[0].astype(jnp.int32)
    dst = edge_index[1].astype(jnp.int32)
    feat_p = jnp.pad(feat, ((0, 8), (0, 0)))
    srcp = jnp.concatenate(
        [src, jnp.full((E_PAD,), N_NODES, jnp.int32)]).reshape(NW, NCH, CH)
    dstp = jnp.concatenate(
        [dst, jnp.arange(E_PAD, dtype=jnp.int32) % NP]).reshape(NW, NCH, CH)
    srcp = jnp.pad(srcp, ((0, 0), (0, 2), (0, 0)),
                   constant_values=N_NODES)
    dstp = jnp.pad(dstp, ((0, 0), (0, 2), (0, 0)))
    parts, deg_parts = _sc_aggregate(feat_p, srcp, dstp)
    return _tc_epilogue(feat, parts, deg_parts.T, W_self, W_neigh,
                        b.reshape(1, D_OUT))


# speedup vs baseline: 2.4800x; 2.3749x over previous
import jax
import jax.numpy as jnp
from jax import lax
from jax.experimental import pallas as pl
from jax.experimental.pallas import tpu as pltpu
from jax.experimental.pallas import tpu_sc as plsc

N_NODES = 10000
N_EDGES = 320000
D_IN = 128
D_OUT = 128

NC = 2
NS = 16
NW = NC * NS
CH = 128
NCH = 80
EPW = NCH * CH
E_PAD = EPW * NW - N_EDGES
NP = 10016
RPT = 624
REM = NP - NS * RPT
ZR = 48


def _sc_aggregate_body(feat_hbm, srcp_hbm, dstp_hbm, parts_hbm, degp_hbm,
                       s0, s1, s2, s3, d0, d1, d2, d3, jidx,
                       rows0, rows1, deg_v, agg,
                       i0, i1, i2, i3, gsem0, gsem1, ssem0, ssem1):
    cid = lax.axis_index("c")
    sid = lax.axis_index("s")
    wid = cid * NS + sid

    zvec = jnp.zeros((16,), jnp.float32)
    jvec = jnp.full((16,), N_NODES, jnp.int32)
    for j in range(CH // 16):
        jidx[pl.ds(j * 16, 16)] = jvec

    def _zero_row(i, _):
        for j in range(D_IN // 16):
            rows0[i, pl.ds(j * 16, 16)] = zvec
        return 0

    lax.fori_loop(0, ZR, _zero_row, 0)

    def _zero_deg(i, _):
        deg_v[pl.ds(i * 16, 16)] = zvec
        return 0

    lax.fori_loop(0, NP // 16, _zero_deg, 0)

    for k in range(RPT // ZR):
        pltpu.sync_copy(rows0.at[pl.ds(0, ZR)],
                        agg.at[pl.ds(sid * RPT + k * ZR, ZR)])

    @pl.when(sid == NS - 1)
    def _zero_rem():
        pltpu.sync_copy(rows0.at[pl.ds(0, REM)], agg.at[pl.ds(NS * RPT, REM)])

    plsc.subcore_barrier()

    ones16 = jnp.ones((16,), jnp.float32)

    def _chunk(c, _):
        pltpu.sync_copy(srcp_hbm.at[wid, c], s0)
        pltpu.sync_copy(dstp_hbm.at[wid, c], d0)
        gat = pltpu.async_copy(feat_hbm.at[s0], rows0, gsem0)
        for g in range(CH // 16):
            d16 = d0[pl.ds(g * 16, 16)]
            s16 = s0[pl.ds(g * 16, 16)]
            plsc.addupdate_scatter(deg_v, [d16], ones16, mask=s16 < N_NODES)
        gat.wait()
        pltpu.sync_copy(rows0, agg.at[d0], add=True)
        return 0

    lax.fori_loop(0, NCH, _chunk, 0)

    plsc.subcore_barrier()

    pltpu.sync_copy(agg.at[pl.ds(sid * RPT, RPT)],
                    parts_hbm.at[cid, pl.ds(sid * RPT, RPT)])

    @pl.when(sid == NS - 1)
    def _write_rem():
        pltpu.sync_copy(agg.at[pl.ds(NS * RPT, REM)],
                        parts_hbm.at[cid, pl.ds(NS * RPT, REM)])

    pltpu.sync_copy(deg_v, degp_hbm.at[wid])


def _sc_aggregate(feat_p, srcp, dstp):
    mesh = plsc.VectorSubcoreMesh(core_axis_name="c", subcore_axis_name="s")
    return pl.kernel(
        _sc_aggregate_body,
        out_type=(jax.ShapeDtypeStruct((NC, NP, D_IN), jnp.float32),
                  jax.ShapeDtypeStruct((NW, NP), jnp.float32)),
        mesh=mesh,
        compiler_params=pltpu.CompilerParams(needs_layout_passes=False),
        scratch_types=[
            pltpu.VMEM((CH,), jnp.int32),
            pltpu.VMEM((CH,), jnp.int32),
            pltpu.VMEM((CH,), jnp.int32),
            pltpu.VMEM((CH,), jnp.int32),
            pltpu.VMEM((CH,), jnp.int32),
            pltpu.VMEM((CH,), jnp.int32),
            pltpu.VMEM((CH,), jnp.int32),
            pltpu.VMEM((CH,), jnp.int32),
            pltpu.VMEM((CH,), jnp.int32),
            pltpu.VMEM((CH, D_IN), jnp.float32),
            pltpu.VMEM((CH, D_IN), jnp.float32),
            pltpu.VMEM((NP,), jnp.float32),
            pltpu.VMEM_SHARED((NP, D_IN), jnp.float32),
            pltpu.SemaphoreType.DMA,
            pltpu.SemaphoreType.DMA,
            pltpu.SemaphoreType.DMA,
            pltpu.SemaphoreType.DMA,
            pltpu.SemaphoreType.DMA,
            pltpu.SemaphoreType.DMA,
            pltpu.SemaphoreType.DMA,
            pltpu.SemaphoreType.DMA,
        ],
    )(feat_p, srcp, dstp)


BR = 1000


def _tc_epilogue_body(feat_ref, parts_ref, degp_ref, ws_ref, wn_ref, b_ref,
                      out_ref):
    agg = parts_ref[0] + parts_ref[1]
    deg = jnp.sum(degp_ref[...], axis=1)[:, None]
    h_neigh = agg / jnp.maximum(deg, 1.0)
    acc = jnp.dot(feat_ref[...], ws_ref[...], preferred_element_type=jnp.float32)
    acc = acc + jnp.dot(h_neigh, wn_ref[...], preferred_element_type=jnp.float32)
    out_ref[...] = jnp.maximum(acc + b_ref[...], 0.0)


def _tc_epilogue(feat, parts, deg_parts_t, W_self, W_neigh, b2d):
    return pl.pallas_call(
        _tc_epilogue_body,
        grid=(N_NODES // BR,),
        in_specs=[
            pl.BlockSpec((BR, D_IN), lambda i: (i, 0)),
            pl.BlockSpec((NC, BR, D_IN), lambda i: (0, i, 0)),
            pl.BlockSpec((BR, NW), lambda i: (i, 0)),
            pl.BlockSpec((D_IN, D_OUT), lambda i: (0, 0)),
            pl.BlockSpec((D_IN, D_OUT), lambda i: (0, 0)),
            pl.BlockSpec((1, D_OUT), lambda i: (0, 0)),
        ],
        out_specs=pl.BlockSpec((BR, D_OUT), lambda i: (i, 0)),
        out_shape=jax.ShapeDtypeStruct((N_NODES, D_OUT), jnp.float32),
    )(feat, parts, deg_parts_t, W_self, W_neigh, b2d)


@jax.jit
def kernel(feat, edge_index, W_self, W_neigh, b):
    src = edge_index[0].astype(jnp.int32)
    dst = edge_index[1].astype(jnp.int32)
    ppw = EPW - N_EDGES // NW
    feat_p = jnp.pad(feat, ((0, 256), (0, 0)))
    pad_s = N_NODES + jnp.arange(NW * ppw, dtype=jnp.int32) % 256
    pad_d = jnp.arange(NW * ppw, dtype=jnp.int32) % N_NODES
    srcp = jnp.concatenate(
        [src.reshape(NW, N_EDGES // NW), pad_s.reshape(NW, ppw)],
        axis=1).reshape(NW, NCH, CH)
    dstp = jnp.concatenate(
        [dst.reshape(NW, N_EDGES // NW), pad_d.reshape(NW, ppw)],
        axis=1).reshape(NW, NCH, CH)
    srcp = jnp.pad(srcp, ((0, 0), (0, 2), (0, 0)),
                   constant_values=N_NODES)
    dstp = jnp.pad(dstp, ((0, 0), (0, 2), (0, 0)))
    parts, deg_parts = _sc_aggregate(feat_p, srcp, dstp)
    return _tc_epilogue(feat, parts, deg_parts.T, W_self, W_neigh,
                        b.reshape(1, D_OUT))


# speedup vs baseline: 2.7301x; 1.1009x over previous
import jax
import jax.numpy as jnp
from jax import lax
from jax.experimental import pallas as pl
from jax.experimental.pallas import tpu as pltpu
from jax.experimental.pallas import tpu_sc as plsc

N_NODES = 10000
N_EDGES = 320000
D_IN = 128
D_OUT = 128

NC = 2
NS = 16
NW = NC * NS
CH = 96
NCH = 106
EPW = NCH * CH
PPW = EPW - N_EDGES // NW
NCHP = 112
NP = 10112
RPT = NP // NS
ZR = 96


def _sc_aggregate_body(feat_hbm, srcp_hbm, dstp_hbm, parts_hbm, degp_hbm,
                       didx, sidx0, sidx1, rows0, rows1, deg_v, agg,
                       isem0, isem1, gsem0, gsem1, ssem0, ssem1):
    cid = lax.axis_index("c")
    sid = lax.axis_index("s")
    wid = cid * NS + sid

    pltpu.sync_copy(dstp_hbm.at[wid], didx)

    zvec = jnp.zeros((16,), jnp.float32)

    def _zero_row(i, _):
        for j in range(D_IN // 16):
            rows0[i, pl.ds(j * 16, 16)] = zvec
        return 0

    lax.fori_loop(0, ZR, _zero_row, 0)

    def _zero_deg(i, _):
        deg_v[pl.ds(i * 16, 16)] = zvec
        return 0

    lax.fori_loop(0, NP // 16, _zero_deg, 0)

    for k in range(RPT // ZR):
        pltpu.sync_copy(rows0.at[pl.ds(0, ZR)],
                        agg.at[pl.ds(sid * RPT + k * ZR, ZR)])
    pltpu.sync_copy(rows0.at[pl.ds(0, RPT % ZR)],
                    agg.at[pl.ds(sid * RPT + (RPT // ZR) * ZR, RPT % ZR)])

    plsc.subcore_barrier()

    ones16 = jnp.ones((16,), jnp.float32)

    pltpu.sync_copy(srcp_hbm.at[wid, 0], sidx0)
    pltpu.async_copy(srcp_hbm.at[wid, 1], sidx1, isem1)
    pltpu.async_copy(rows0, agg.at[didx.at[NCH]], ssem1, add=True)
    pltpu.async_copy(feat_hbm.at[sidx0], rows0, gsem0)

    def _pair(t, _):
        for (b, sidx, isem, rows, gsem, ssem,
             sidx_n, isem_n, rows_n, gsem_n, ssem_n) in (
                (0, sidx0, isem0, rows0, gsem0, ssem0,
                 sidx1, isem1, rows1, gsem1, ssem1),
                (1, sidx1, isem1, rows1, gsem1, ssem1,
                 sidx0, isem0, rows0, gsem0, ssem0)):
            c = 2 * t + b
            for g in range(CH // 16):
                d16 = didx[c, pl.ds(g * 16, 16)]
                plsc.addupdate_scatter(deg_v, [d16], ones16)
            pltpu.make_async_copy(srcp_hbm.at[wid, c], sidx_n, isem_n).wait()
            pltpu.make_async_copy(rows_n, agg.at[didx.at[NCH]], ssem_n).wait()
            pltpu.async_copy(feat_hbm.at[sidx_n], rows_n, gsem_n)
            pltpu.make_async_copy(feat_hbm.at[sidx_n], rows, gsem).wait()
            pltpu.async_copy(srcp_hbm.at[wid, c + 2], sidx, isem)
            pltpu.async_copy(rows, agg.at[didx.at[c]], ssem, add=True)
        return 0

    lax.fori_loop(0, NCH // 2, _pair, 0)

    pltpu.make_async_copy(feat_hbm.at[sidx0], rows0, gsem0).wait()
    pltpu.make_async_copy(rows1, agg.at[didx.at[NCH]], ssem1).wait()
    pltpu.make_async_copy(srcp_hbm.at[wid, 0], sidx1, isem1).wait()

    plsc.subcore_barrier()

    pltpu.sync_copy(agg.at[pl.ds(sid * RPT, RPT)],
                    parts_hbm.at[cid, pl.ds(sid * RPT, RPT)])
    pltpu.sync_copy(deg_v, degp_hbm.at[wid])


def _sc_aggregate(feat, srcp, dstp):
    mesh = plsc.VectorSubcoreMesh(core_axis_name="c", subcore_axis_name="s")
    return pl.kernel(
        _sc_aggregate_body,
        out_type=(jax.ShapeDtypeStruct((NC, NP, D_IN), jnp.float32),
                  jax.ShapeDtypeStruct((NW, NP), jnp.float32)),
        mesh=mesh,
        compiler_params=pltpu.CompilerParams(needs_layout_passes=False),
        scratch_types=[
            pltpu.VMEM((NCHP, CH), jnp.int32),
            pltpu.VMEM((CH,), jnp.int32),
            pltpu.VMEM((CH,), jnp.int32),
            pltpu.VMEM((CH, D_IN), jnp.float32),
            pltpu.VMEM((CH, D_IN), jnp.float32),
            pltpu.VMEM((NP,), jnp.float32),
            pltpu.VMEM_SHARED((NP, D_IN), jnp.float32),
            pltpu.SemaphoreType.DMA,
            pltpu.SemaphoreType.DMA,
            pltpu.SemaphoreType.DMA,
            pltpu.SemaphoreType.DMA,
            pltpu.SemaphoreType.DMA,
            pltpu.SemaphoreType.DMA,
        ],
    )(feat, srcp, dstp)


BR = 1000


def _tc_epilogue_body(feat_ref, parts_ref, degp_ref, ws_ref, wn_ref, b_ref,
                      out_ref):
    agg = parts_ref[0] + parts_ref[1]
    deg = jnp.sum(degp_ref[...], axis=1)[:, None]
    h_neigh = agg / jnp.maximum(deg, 1.0)
    acc = jnp.dot(feat_ref[...], ws_ref[...], preferred_element_type=jnp.float32)
    acc = acc + jnp.dot(h_neigh, wn_ref[...], preferred_element_type=jnp.float32)
    out_ref[...] = jnp.maximum(acc + b_ref[...], 0.0)


def _tc_epilogue(feat, parts, deg_parts_t, W_self, W_neigh, b2d):
    return pl.pallas_call(
        _tc_epilogue_body,
        grid=(N_NODES // BR,),
        in_specs=[
            pl.BlockSpec((BR, D_IN), lambda i: (i, 0)),
            pl.BlockSpec((NC, BR, D_IN), lambda i: (0, i, 0)),
            pl.BlockSpec((BR, NW), lambda i: (i, 0)),
            pl.BlockSpec((D_IN, D_OUT), lambda i: (0, 0)),
            pl.BlockSpec((D_IN, D_OUT), lambda i: (0, 0)),
            pl.BlockSpec((1, D_OUT), lambda i: (0, 0)),
        ],
        out_specs=pl.BlockSpec((BR, D_OUT), lambda i: (i, 0)),
        out_shape=jax.ShapeDtypeStruct((N_NODES, D_OUT), jnp.float32),
    )(feat, parts, deg_parts_t, W_self, W_neigh, b2d)


@jax.jit
def kernel(feat, edge_index, W_self, W_neigh, b):
    src = edge_index[0].astype(jnp.int32)
    dst = edge_index[1].astype(jnp.int32)
    pad_s = (jnp.arange(NW * PPW, dtype=jnp.int32) * 37) % N_NODES
    pad_d = N_NODES + jnp.arange(NW * PPW, dtype=jnp.int32) % (NP - N_NODES)
    srcp = jnp.concatenate(
        [src.reshape(NW, N_EDGES // NW), pad_s.reshape(NW, PPW)],
        axis=1).reshape(NW, NCH, CH)
    dstp = jnp.concatenate(
        [dst.reshape(NW, N_EDGES // NW), pad_d.reshape(NW, PPW)],
        axis=1).reshape(NW, NCH, CH)
    srcp = jnp.pad(srcp, ((0, 0), (0, NCHP - NCH), (0, 0)))
    junk_rows = N_NODES + (jnp.arange((NCHP - NCH) * CH, dtype=jnp.int32)
                           % (NP - N_NODES))
    dstp = jnp.concatenate(
        [dstp, jnp.broadcast_to(junk_rows.reshape(1, NCHP - NCH, CH),
                                (NW, NCHP - NCH, CH))], axis=1)
    parts, deg_parts = _sc_aggregate(feat, srcp, dstp)
    return _tc_epilogue(feat, parts, deg_parts.T, W_self, W_neigh,
                        b.reshape(1, D_OUT))


# speedup vs baseline: 5.0297x; 1.8423x over previous
import jax
import jax.numpy as jnp
from jax import lax
from jax.experimental import pallas as pl
from jax.experimental.pallas import tpu as pltpu
from jax.experimental.pallas import tpu_sc as plsc

N_NODES = 10000
N_EDGES = 320000
D_IN = 128
D_OUT = 128

NC = 2
NS = 16
NW = NC * NS
EPW = N_EDGES // NW
CH = 96
NCH = 104
TAIL = EPW - NCH * CH
RPT = 624
REM = N_NODES - NS * RPT
ZR = 96


def _sc_aggregate_body(feat_hbm, eflat_hbm, parts_hbm, degp_hbm,
                       s0, s1, s2, s3, d0, d1, d2, d3,
                       rows0, rows1, st, dt, rows_t, deg_v, agg,
                       i0, i1, i2, i3, gsem0, gsem1, ssem0, ssem1):
    cid = lax.axis_index("c")
    sid = lax.axis_index("s")
    wid = cid * NS + sid
    sbase = wid * EPW
    dbase = N_EDGES + wid * EPW

    zvec = jnp.zeros((16,), jnp.float32)

    def _zero_row(i, _):
        for j in range(D_IN // 16):
            rows0[i, pl.ds(j * 16, 16)] = zvec
            rows1[i, pl.ds(j * 16, 16)] = zvec
        return 0

    lax.fori_loop(0, ZR, _zero_row, 0)

    def _zero_deg(i, _):
        deg_v[pl.ds(i * 16, 16)] = zvec
        return 0

    lax.fori_loop(0, N_NODES // 16, _zero_deg, 0)

    for k in range(RPT // ZR):
        pltpu.sync_copy(rows0.at[pl.ds(0, ZR)],
                        agg.at[pl.ds(sid * RPT + k * ZR, ZR)])
    pltpu.sync_copy(rows0.at[pl.ds(0, RPT - (RPT // ZR) * ZR)],
                    agg.at[pl.ds(sid * RPT + (RPT // ZR) * ZR,
                                 RPT - (RPT // ZR) * ZR)])

    @pl.when(sid == NS - 1)
    def _zero_rem():
        pltpu.sync_copy(rows0.at[pl.ds(0, REM)], agg.at[pl.ds(NS * RPT, REM)])

    plsc.subcore_barrier()

    ones16 = jnp.ones((16,), jnp.float32)
    idxs = ((s0, d0, i0), (s1, d1, i1), (s2, d2, i2), (s3, d3, i3))
    rbuf = ((rows0, gsem0, ssem0), (rows1, gsem1, ssem1))

    pltpu.sync_copy(eflat_hbm.at[pl.ds(sbase, CH)], s0)
    pltpu.sync_copy(eflat_hbm.at[pl.ds(dbase, CH)], d0)
    pltpu.async_copy(eflat_hbm.at[pl.ds(sbase + CH, CH)], s1, i1)
    pltpu.async_copy(eflat_hbm.at[pl.ds(dbase + CH, CH)], d1, i1)
    pltpu.async_copy(rows1, agg.at[d0], ssem1, add=True)
    pltpu.async_copy(feat_hbm.at[s0], rows0, gsem0)

    def _quad(t, _):
        for b in range(4):
            c = 4 * t + b
            sidx_c, didx_c, _ = idxs[b]
            sidx_n, didx_n, isem_n = idxs[(b + 1) % 4]
            sidx_l, didx_l, isem_l = idxs[(b + 2) % 4]
            rows_p, gsem_p, ssem_p = rbuf[b % 2]
            rows_q, gsem_q, ssem_q = rbuf[1 - b % 2]
            for g in range(CH // 16):
                d16 = didx_c[pl.ds(g * 16, 16)]
                plsc.addupdate_scatter(deg_v, [d16], ones16)
            cpre = jnp.minimum(c + 2, NCH - 1)
            pltpu.async_copy(eflat_hbm.at[pl.ds(sbase + cpre * CH, CH)],
                             sidx_l, isem_l)
            pltpu.async_copy(eflat_hbm.at[pl.ds(dbase + cpre * CH, CH)],
                             didx_l, isem_l)
            pltpu.make_async_copy(rows_q, agg.at[didx_c], ssem_q).wait()
            pltpu.make_async_copy(eflat_hbm.at[pl.ds(sbase, CH)],
                                  sidx_n, isem_n).wait()
            pltpu.make_async_copy(eflat_hbm.at[pl.ds(dbase, CH)],
                                  didx_n, isem_n).wait()
            pltpu.async_copy(feat_hbm.at[sidx_n], rows_q, gsem_q)
            pltpu.make_async_copy(feat_hbm.at[sidx_c], rows_p, gsem_p).wait()
            pltpu.async_copy(rows_p, agg.at[didx_c], ssem_p, add=True)
        return 0

    lax.fori_loop(0, NCH // 4, _quad, 0)

    pltpu.make_async_copy(feat_hbm.at[s0], rows0, gsem0).wait()
    pltpu.make_async_copy(rows1, agg.at[d0], ssem1).wait()
    pltpu.make_async_copy(eflat_hbm.at[pl.ds(sbase, CH)], s1, i1).wait()
    pltpu.make_async_copy(eflat_hbm.at[pl.ds(dbase, CH)], d1, i1).wait()

    pltpu.sync_copy(eflat_hbm.at[pl.ds(sbase + NCH * CH, TAIL)], st)
    pltpu.sync_copy(eflat_hbm.at[pl.ds(dbase + NCH * CH, TAIL)], dt)
    pltpu.async_copy(feat_hbm.at[st], rows_t, gsem0).wait()
    d16 = dt[pl.ds(0, TAIL)]
    plsc.addupdate_scatter(deg_v, [d16], ones16)
    pltpu.sync_copy(rows_t, agg.at[dt], add=True)

    plsc.subcore_barrier()

    pltpu.sync_copy(agg.at[pl.ds(sid * RPT, RPT)],
                    parts_hbm.at[cid, pl.ds(sid * RPT, RPT)])

    @pl.when(sid == NS - 1)
    def _write_rem():
        pltpu.sync_copy(agg.at[pl.ds(NS * RPT, REM)],
                        parts_hbm.at[cid, pl.ds(NS * RPT, REM)])

    pltpu.sync_copy(deg_v, degp_hbm.at[wid])


def _sc_aggregate(feat, eflat):
    mesh = plsc.VectorSubcoreMesh(core_axis_name="c", subcore_axis_name="s")
    return pl.kernel(
        _sc_aggregate_body,
        out_type=(jax.ShapeDtypeStruct((NC, N_NODES, D_IN), jnp.float32),
                  jax.ShapeDtypeStruct((NW, N_NODES), jnp.float32)),
        mesh=mesh,
        compiler_params=pltpu.CompilerParams(needs_layout_passes=False),
        scratch_types=[
            pltpu.VMEM((CH,), jnp.int32),
            pltpu.VMEM((CH,), jnp.int32),
            pltpu.VMEM((CH,), jnp.int32),
            pltpu.VMEM((CH,), jnp.int32),
            pltpu.VMEM((CH,), jnp.int32),
            pltpu.VMEM((CH,), jnp.int32),
            pltpu.VMEM((CH,), jnp.int32),
            pltpu.VMEM((CH,), jnp.int32),
            pltpu.VMEM((CH, D_IN), jnp.float32),
            pltpu.VMEM((CH, D_IN), jnp.float32),
            pltpu.VMEM((TAIL,), jnp.int32),
            pltpu.VMEM((TAIL,), jnp.int32),
            pltpu.VMEM((TAIL, D_IN), jnp.float32),
            pltpu.VMEM((N_NODES,), jnp.float32),
            pltpu.VMEM_SHARED((N_NODES, D_IN), jnp.float32),
            pltpu.SemaphoreType.DMA,
            pltpu.SemaphoreType.DMA,
            pltpu.SemaphoreType.DMA,
            pltpu.SemaphoreType.DMA,
            pltpu.SemaphoreType.DMA,
            pltpu.SemaphoreType.DMA,
            pltpu.SemaphoreType.DMA,
            pltpu.SemaphoreType.DMA,
        ],
    )(feat, eflat)


BR = 1000


def _tc_epilogue_body(feat_ref, parts_ref, degp_ref, ws_ref, wn_ref, b_ref,
                      out_ref):
    agg = parts_ref[0] + parts_ref[1]
    deg = jnp.sum(degp_ref[...], axis=1)[:, None]
    h_neigh = agg / jnp.maximum(deg, 1.0)
    acc = jnp.dot(feat_ref[...], ws_ref[...], preferred_element_type=jnp.float32)
    acc = acc + jnp.dot(h_neigh, wn_ref[...], preferred_element_type=jnp.float32)
    out_ref[...] = jnp.maximum(acc + b_ref[...], 0.0)


def _tc_epilogue(feat, parts, deg_parts_t, W_self, W_neigh, b2d):
    return pl.pallas_call(
        _tc_epilogue_body,
        grid=(N_NODES // BR,),
        in_specs=[
            pl.BlockSpec((BR, D_IN), lambda i: (i, 0)),
            pl.BlockSpec((NC, BR, D_IN), lambda i: (0, i, 0)),
            pl.BlockSpec((BR, NW), lambda i: (i, 0)),
            pl.BlockSpec((D_IN, D_OUT), lambda i: (0, 0)),
            pl.BlockSpec((D_IN, D_OUT), lambda i: (0, 0)),
            pl.BlockSpec((1, D_OUT), lambda i: (0, 0)),
        ],
        out_specs=pl.BlockSpec((BR, D_OUT), lambda i: (i, 0)),
        out_shape=jax.ShapeDtypeStruct((N_NODES, D_OUT), jnp.float32),
    )(feat, parts, deg_parts_t, W_self, W_neigh, b2d)


@jax.jit
def kernel(feat, edge_index, W_self, W_neigh, b):
    eflat = edge_index.astype(jnp.int32).reshape(2 * N_EDGES)
    parts, deg_parts = _sc_aggregate(feat, eflat)
    return _tc_epilogue(feat, parts, deg_parts.T, W_self, W_neigh,
                        b.reshape(1, D_OUT))


# speedup vs baseline: 5.1067x; 1.0153x over previous
import jax
import jax.numpy as jnp
from jax import lax
from jax.experimental import pallas as pl
from jax.experimental.pallas import tpu as pltpu
from jax.experimental.pallas import tpu_sc as plsc

N_NODES = 10000
N_EDGES = 320000
D_IN = 128
D_OUT = 128

NC = 2
NS = 16
NW = NC * NS
EPW = N_EDGES // NW
CH = 104
NCH = 96
TAIL = EPW - NCH * CH
RPT = 624
REM = N_NODES - NS * RPT
ZR = 96


def _sc_aggregate_body(feat_hbm, eflat_hbm, parts_hbm, degp_hbm,
                       s0, s1, s2, s3, d0, d1, d2, d3,
                       rows0, rows1, st, dt, rows_t, deg_v, agg,
                       i0, i1, i2, i3, gsem0, gsem1, ssem0, ssem1):
    cid = lax.axis_index("c")
    sid = lax.axis_index("s")
    wid = cid * NS + sid
    sbase = wid * EPW
    dbase = N_EDGES + wid * EPW

    zvec = jnp.zeros((16,), jnp.float32)

    def _zero_row(i, _):
        for j in range(D_IN // 16):
            rows0[i, pl.ds(j * 16, 16)] = zvec
            rows1[i, pl.ds(j * 16, 16)] = zvec
        return 0

    lax.fori_loop(0, ZR, _zero_row, 0)

    def _zero_deg(i, _):
        deg_v[pl.ds(i * 16, 16)] = zvec
        return 0

    lax.fori_loop(0, N_NODES // 16, _zero_deg, 0)

    for k in range(RPT // ZR):
        pltpu.sync_copy(rows0.at[pl.ds(0, ZR)],
                        agg.at[pl.ds(sid * RPT + k * ZR, ZR)])
    pltpu.sync_copy(rows0.at[pl.ds(0, RPT - (RPT // ZR) * ZR)],
                    agg.at[pl.ds(sid * RPT + (RPT // ZR) * ZR,
                                 RPT - (RPT // ZR) * ZR)])

    @pl.when(sid == NS - 1)
    def _zero_rem():
        pltpu.sync_copy(rows0.at[pl.ds(0, REM)], agg.at[pl.ds(NS * RPT, REM)])

    plsc.subcore_barrier()

    ones16 = jnp.ones((16,), jnp.float32)
    idxs = ((s0, d0, i0), (s1, d1, i1), (s2, d2, i2), (s3, d3, i3))
    rbuf = ((rows0, gsem0, ssem0), (rows1, gsem1, ssem1))

    pltpu.sync_copy(eflat_hbm.at[pl.ds(sbase, CH)], s0)
    pltpu.sync_copy(eflat_hbm.at[pl.ds(dbase, CH)], d0)
    pltpu.async_copy(eflat_hbm.at[pl.ds(sbase + CH, CH)], s1, i1)
    pltpu.async_copy(eflat_hbm.at[pl.ds(dbase + CH, CH)], d1, i1)
    pltpu.async_copy(rows1, agg.at[d0], ssem1, add=True)
    pltpu.async_copy(feat_hbm.at[s0], rows0, gsem0)

    def _quad(t, _):
        for b in range(4):
            c = 4 * t + b
            sidx_c, didx_c, _ = idxs[b]
            sidx_n, didx_n, isem_n = idxs[(b + 1) % 4]
            sidx_l, didx_l, isem_l = idxs[(b + 2) % 4]
            rows_p, gsem_p, ssem_p = rbuf[b % 2]
            rows_q, gsem_q, ssem_q = rbuf[1 - b % 2]
            for g in range(CH // 16):
                d16 = didx_c[pl.ds(g * 16, 16)]
                plsc.addupdate_scatter(deg_v, [d16], ones16)
            cpre = jnp.minimum(c + 2, NCH - 1)
            pltpu.async_copy(eflat_hbm.at[pl.ds(sbase + cpre * CH, CH)],
                             sidx_l, isem_l)
            pltpu.async_copy(eflat_hbm.at[pl.ds(dbase + cpre * CH, CH)],
                             didx_l, isem_l)
            pltpu.make_async_copy(rows_q, agg.at[didx_c], ssem_q).wait()
            pltpu.make_async_copy(eflat_hbm.at[pl.ds(sbase, CH)],
                                  sidx_n, isem_n).wait()
            pltpu.make_async_copy(eflat_hbm.at[pl.ds(dbase, CH)],
                                  didx_n, isem_n).wait()
            pltpu.async_copy(feat_hbm.at[sidx_n], rows_q, gsem_q)
            pltpu.make_async_copy(feat_hbm.at[sidx_c], rows_p, gsem_p).wait()
            pltpu.async_copy(rows_p, agg.at[didx_c], ssem_p, add=True)
        return 0

    lax.fori_loop(0, NCH // 4, _quad, 0)

    pltpu.make_async_copy(feat_hbm.at[s0], rows0, gsem0).wait()
    pltpu.make_async_copy(rows1, agg.at[d0], ssem1).wait()
    pltpu.make_async_copy(eflat_hbm.at[pl.ds(sbase, CH)], s1, i1).wait()
    pltpu.make_async_copy(eflat_hbm.at[pl.ds(dbase, CH)], d1, i1).wait()

    pltpu.sync_copy(eflat_hbm.at[pl.ds(sbase + NCH * CH, TAIL)], st)
    pltpu.sync_copy(eflat_hbm.at[pl.ds(dbase + NCH * CH, TAIL)], dt)
    pltpu.async_copy(feat_hbm.at[st], rows_t, gsem0).wait()
    d16 = dt[pl.ds(0, TAIL)]
    plsc.addupdate_scatter(deg_v, [d16], ones16)
    pltpu.sync_copy(rows_t, agg.at[dt], add=True)

    plsc.subcore_barrier()

    pltpu.sync_copy(agg.at[pl.ds(sid * RPT, RPT)],
                    parts_hbm.at[cid, pl.ds(sid * RPT, RPT)])

    @pl.when(sid == NS - 1)
    def _write_rem():
        pltpu.sync_copy(agg.at[pl.ds(NS * RPT, REM)],
                        parts_hbm.at[cid, pl.ds(NS * RPT, REM)])

    pltpu.sync_copy(deg_v, degp_hbm.at[wid])


def _sc_aggregate(feat, eflat):
    mesh = plsc.VectorSubcoreMesh(core_axis_name="c", subcore_axis_name="s")
    return pl.kernel(
        _sc_aggregate_body,
        out_type=(jax.ShapeDtypeStruct((NC, N_NODES, D_IN), jnp.float32),
                  jax.ShapeDtypeStruct((NW, N_NODES), jnp.float32)),
        mesh=mesh,
        compiler_params=pltpu.CompilerParams(needs_layout_passes=False),
        scratch_types=[
            pltpu.VMEM((CH,), jnp.int32),
            pltpu.VMEM((CH,), jnp.int32),
            pltpu.VMEM((CH,), jnp.int32),
            pltpu.VMEM((CH,), jnp.int32),
            pltpu.VMEM((CH,), jnp.int32),
            pltpu.VMEM((CH,), jnp.int32),
            pltpu.VMEM((CH,), jnp.int32),
            pltpu.VMEM((CH,), jnp.int32),
            pltpu.VMEM((CH, D_IN), jnp.float32),
            pltpu.VMEM((CH, D_IN), jnp.float32),
            pltpu.VMEM((TAIL,), jnp.int32),
            pltpu.VMEM((TAIL,), jnp.int32),
            pltpu.VMEM((TAIL, D_IN), jnp.float32),
            pltpu.VMEM((N_NODES,), jnp.float32),
            pltpu.VMEM_SHARED((N_NODES, D_IN), jnp.float32),
            pltpu.SemaphoreType.DMA,
            pltpu.SemaphoreType.DMA,
            pltpu.SemaphoreType.DMA,
            pltpu.SemaphoreType.DMA,
            pltpu.SemaphoreType.DMA,
            pltpu.SemaphoreType.DMA,
            pltpu.SemaphoreType.DMA,
            pltpu.SemaphoreType.DMA,
        ],
    )(feat, eflat)


BR = 1000


def _tc_epilogue_body(feat_ref, parts_ref, degp_ref, ws_ref, wn_ref, b_ref,
                      out_ref):
    agg = parts_ref[0] + parts_ref[1]
    deg = jnp.sum(degp_ref[...], axis=1)[:, None]
    h_neigh = agg / jnp.maximum(deg, 1.0)
    acc = jnp.dot(feat_ref[...], ws_ref[...], preferred_element_type=jnp.float32)
    acc = acc + jnp.dot(h_neigh, wn_ref[...], preferred_element_type=jnp.float32)
    out_ref[...] = jnp.maximum(acc + b_ref[...], 0.0)


def _tc_epilogue(feat, parts, deg_parts_t, W_self, W_neigh, b2d):
    return pl.pallas_call(
        _tc_epilogue_body,
        grid=(N_NODES // BR,),
        in_specs=[
            pl.BlockSpec((BR, D_IN), lambda i: (i, 0)),
            pl.BlockSpec((NC, BR, D_IN), lambda i: (0, i, 0)),
            pl.BlockSpec((BR, NW), lambda i: (i, 0)),
            pl.BlockSpec((D_IN, D_OUT), lambda i: (0, 0)),
            pl.BlockSpec((D_IN, D_OUT), lambda i: (0, 0)),
            pl.BlockSpec((1, D_OUT), lambda i: (0, 0)),
        ],
        out_specs=pl.BlockSpec((BR, D_OUT), lambda i: (i, 0)),
        out_shape=jax.ShapeDtypeStruct((N_NODES, D_OUT), jnp.float32),
    )(feat, parts, deg_parts_t, W_self, W_neigh, b2d)


@jax.jit
def kernel(feat, edge_index, W_self, W_neigh, b):
    eflat = edge_index.astype(jnp.int32).reshape(2 * N_EDGES)
    parts, deg_parts = _sc_aggregate(feat, eflat)
    return _tc_epilogue(feat, parts, deg_parts.T, W_self, W_neigh,
                        b.reshape(1, D_OUT))
